# baseline (speedup 1.0000x reference)
import jax
import jax.numpy as jnp
from jax.experimental import pallas as pl
from jax.experimental.pallas import tpu as pltpu
from jax.experimental.pallas import tpu_sc as plsc

H = 1024
NCHUNK = 4
KC = H // NCHUNK


def _gather_sc(idx_hbm, emb_hbm, x_hbm, idx_smem, sem):
    core = jax.lax.axis_index("core")

    @pl.when(core == 0)
    def _():
        pltpu.async_copy(idx_hbm, idx_smem, sem).wait()
        i = idx_smem[0]
        pltpu.async_copy(emb_hbm.at[pl.ds(i, 1), :], x_hbm, sem).wait()


def _sru_body(x_ref, W_hbm, h_ref, c_ref, W_vmem, sem_w):
    copies = []
    for i in range(NCHUNK):
        cp = pltpu.make_async_copy(
            W_hbm.at[pl.ds(i * KC, KC), :],
            W_vmem.at[pl.ds(i * KC, KC), :],
            sem_w.at[i],
        )
        cp.start()
        copies.append(cp)
    x = x_ref[...]
    u = None
    for i in range(NCHUNK):
        copies[i].wait()
        ui = jax.lax.dot_general(
            x[:, i * KC:(i + 1) * KC],
            W_vmem[pl.ds(i * KC, KC), :],
            (((1,), (0,)), ((), ())),
            preferred_element_type=jnp.float32,
        )
        u = ui if u is None else u + ui
    x_t = u[:, :H]
    f = jax.nn.sigmoid(u[:, H:2 * H])
    r = jax.nn.sigmoid(u[:, 2 * H:])
    c = (1.0 - f) * x_t
    h = r * jnp.tanh(c) + (1.0 - r) * x
    h_ref[0] = h
    c_ref[0] = c


def kernel(input, hidden, cell, emb, W, b_f, b_r):
    idx = input.astype(jnp.int32)
    gather = pl.kernel(
        _gather_sc,
        out_type=jax.ShapeDtypeStruct((1, H), jnp.float32),
        mesh=plsc.ScalarSubcoreMesh(axis_name="core", num_cores=2),
        scratch_types=[pltpu.SMEM((1,), jnp.int32), pltpu.SemaphoreType.DMA],
    )
    x = gather(idx, emb)
    h, c = pl.pallas_call(
        _sru_body,
        in_specs=[
            pl.BlockSpec((1, H), lambda: (0, 0)),
            pl.BlockSpec(memory_space=pltpu.MemorySpace.HBM),
        ],
        out_specs=[
            pl.BlockSpec((1, 1, H), lambda: (0, 0, 0)),
            pl.BlockSpec((1, 1, H), lambda: (0, 0, 0)),
        ],
        scratch_shapes=[
            pltpu.VMEM((H, 3 * H), jnp.float32),
            pltpu.SemaphoreType.DMA((NCHUNK,)),
        ],
        out_shape=[
            jax.ShapeDtypeStruct((1, 1, H), jnp.float32),
            jax.ShapeDtypeStruct((1, 1, H), jnp.float32),
        ],
    )(x, W)
    return h, c
